# baseline (speedup 1.0000x reference)
import functools

import jax
import jax.numpy as jnp
from jax import lax
from jax.experimental import pallas as pl
from jax.experimental.pallas import tpu as pltpu
from jax.experimental.pallas import tpu_sc as plsc

_B, _NOBJ, _K, _C, _H, _W = 32, 32, 7, 24, 192, 192
_HW = _H * _W
_CHW = _C * _HW
_SUB = _CHW // 128
_K2 = 2 * _K
_OFFSET_W_RATIO = 1.0
_OFFSET_H_RATIO = 1.0
_HH = 1.0
_EPS = 1e-07


def _argmax_body(hm_ref, out_ref):
    vc = hm_ref[0]
    cm = jnp.max(jnp.max(vc, axis=1), axis=1)
    m = jnp.max(cm)
    c_iota = lax.broadcasted_iota(jnp.int32, (_C,), 0)
    cstar = jnp.min(jnp.where(cm == m, c_iota, jnp.int32(_C)))
    blk = hm_ref[0, cstar]
    row = lax.broadcasted_iota(jnp.int32, (_H, _W), 0)
    col = lax.broadcasted_iota(jnp.int32, (_H, _W), 1)
    flat = row * _W + col
    p = jnp.min(jnp.where(blk == m, flat, jnp.int32(_HW)))
    out_ref[0] = jnp.full((1, 8), p, jnp.int32)


def _rsqrt(x):
    i = lax.bitcast_convert_type(x, jnp.int32)
    i = jnp.int32(0x5F3759DF) - lax.shift_right_arithmetic(i, 1)
    y = lax.bitcast_convert_type(i, jnp.float32)
    for _ in range(4):
        y = y * (1.5 - 0.5 * x * y * y)
    return y


def _take16(v, idx):
    return lax.gather(
        v, idx[:, None],
        dimension_numbers=lax.GatherDimensionNumbers(
            offset_dims=(), collapsed_slice_dims=(0,), start_index_map=(0,)),
        slice_sizes=(1,),
        mode=lax.GatherScatterMode.PROMISE_IN_BOUNDS)


def _sc_loss_body(wh_hbm, off_hbm, ckp_hbm, tgt_hbm, msk_hbm, p_hbm, i0_hbm,
                  out_hbm,
                  wh_tile, off_tile, ckp_v, tgt_v, msk_v, p_v, i0_v, res_v):
    b = lax.axis_index("s") * 2 + lax.axis_index("c")
    lane = lax.broadcasted_iota(jnp.int32, (16,), 0)
    bm = lane == (b % 16)

    pltpu.sync_copy(p_hbm, p_v)
    pltpu.sync_copy(i0_hbm, i0_v)
    pltpu.sync_copy(ckp_hbm, ckp_v)
    pltpu.sync_copy(tgt_hbm, tgt_v)
    pltpu.sync_copy(msk_hbm, msk_v)

    def _scalar_i32(ref):
        e0 = jnp.max(jnp.where(bm, ref[pl.ds(0, 16)], jnp.int32(-1)))
        e1 = jnp.max(jnp.where(bm, ref[pl.ds(16, 16)], jnp.int32(-1)))
        return jnp.where(b < 16, e0, e1)

    pp = _scalar_i32(p_v)
    i0 = _scalar_i32(i0_v)
    y1, x1 = i0 // _W, i0 % _W
    y2, x2 = pp // _W, pp % _W
    pltpu.sync_copy(
        wh_hbm.at[b, :, pl.ds(8 * (y1 // 8), 8), pl.ds(128 * (x1 // 128), 128)],
        wh_tile)
    pltpu.sync_copy(
        off_hbm.at[b, :, pl.ds(8 * (y2 // 8), 8), pl.ds(128 * (x2 // 128), 128)],
        off_tile)

    c14 = jnp.minimum(lane, 13)
    bb = jnp.full((16,), b, jnp.int32)
    pred = plsc.load_gather(wh_tile, [c14, jnp.full((16,), y1 % 8, jnp.int32),
                                      jnp.full((16,), x1 % 128, jnp.int32)])
    off = plsc.load_gather(off_tile, [c14, jnp.full((16,), y2 % 8, jnp.int32),
                                      jnp.full((16,), x2 % 128, jnp.int32)])
    mskv = plsc.load_gather(msk_v, [bb])
    ckp = plsc.load_gather(ckp_v, [bb, c14]) * mskv
    tgt = plsc.load_gather(tgt_v, [bb, c14]) * mskv

    xs = x2.astype(jnp.float32)
    ys = y2.astype(jnp.float32)
    pos = jnp.where(lane % 2 == 0, jnp.full((16,), xs), jnp.full((16,), ys))
    p1 = pos + off - pred * 0.5
    p2 = pos + off + pred * 0.5
    g1 = ckp - tgt * 0.5
    g2 = ckp + tgt * 0.5
    pc = (p1 + p2) / 2
    gc = (g1 + g2) / 2

    idx2 = jnp.minimum(lane + 2, 15)
    idx1 = jnp.minimum(lane + 1, 15)

    def sh2(v):
        return _take16(v, idx2)

    def pair(v):
        return v + _take16(v, idx1)

    vgg = sh2(gc) - gc
    vpp = sh2(pc) - pc
    vgp = sh2(gc) - pc
    vpg = sh2(pc) - gc
    dp = pc - gc
    dn = sh2(pc) - sh2(gc)

    d2p = pair(dp * dp) + _EPS
    d2n = pair(dn * dn) + _EPS
    n2p = pair(vgp * vgp) + _HH
    n2n = pair(vpg * vpg) + _HH
    rp = _rsqrt(n2p)
    rn = _rsqrt(n2n)
    sin = (d2p * _rsqrt(d2p) * rp + d2n * _rsqrt(d2n) * rn) / 2
    cross = (pair(vgp * vpg) + _HH) * (rp * rn)
    own = (pair(vgg * vpp) + _HH) * (
        _rsqrt(pair(vgg * vgg) + _HH) * _rsqrt(pair(vpp * vpp) + _HH))
    cos = 1.0 - (cross + own) / 2
    gsin = _take16(0.5 * sin, jnp.minimum(lane * 2, 15))
    gcos = _take16(0.5 * cos, jnp.clip((lane - 8) * 2, 0, 15))
    res_v[...] = jnp.where(lane < 8, gsin, gcos)
    pltpu.sync_copy(res_v, out_hbm.at[b])


def kernel(centerKpoints, target_wh, output_hm, output_wh, output_STA_offset,
           mask, index):
    p3 = pl.pallas_call(
        _argmax_body,
        grid=(_B,),
        in_specs=[pl.BlockSpec((1, _C, _H, _W), lambda b: (b, 0, 0, 0))],
        out_specs=pl.BlockSpec((1, 1, 8), lambda b: (b, 0, 0)),
        out_shape=jax.ShapeDtypeStruct((_B, 1, 8), jnp.int32),
    )(output_hm)
    p = p3[:, 0, 0]
    idx0 = index[:, 0].astype(jnp.int32)
    ckp0 = centerKpoints[:, 0, :]
    tgt0 = target_wh[:, 0, :]
    msk0 = mask[:, 0]

    sc_loss = pl.kernel(
        _sc_loss_body,
        out_type=jax.ShapeDtypeStruct((_B, 16), jnp.float32),
        mesh=plsc.VectorSubcoreMesh(core_axis_name="c", subcore_axis_name="s",
                                    num_cores=2, num_subcores=16),
        scratch_types=[
            pltpu.VMEM((_K2, 8, 128), jnp.float32),
            pltpu.VMEM((_K2, 8, 128), jnp.float32),
            pltpu.VMEM((_B, _K2), jnp.float32),
            pltpu.VMEM((_B, _K2), jnp.float32),
            pltpu.VMEM((_B,), jnp.float32),
            pltpu.VMEM((_B,), jnp.int32),
            pltpu.VMEM((_B,), jnp.int32),
            pltpu.VMEM((16,), jnp.float32),
        ],
        compiler_params=pltpu.CompilerParams(use_tc_tiling_on_sc=True,
                                             needs_layout_passes=False),
    )
    out = sc_loss(output_wh, output_STA_offset, ckp0, tgt0, msk0, p, idx0)
    sin = out[:, 0:6].reshape(-1)
    cos = out[:, 8:14].reshape(-1)
    return sin, cos


# speedup vs baseline: 1.0361x; 1.0361x over previous
import functools

import jax
import jax.numpy as jnp
from jax import lax
from jax.experimental import pallas as pl
from jax.experimental.pallas import tpu as pltpu
from jax.experimental.pallas import tpu_sc as plsc

_B, _NOBJ, _K, _C, _H, _W = 32, 32, 7, 24, 192, 192
_HW = _H * _W
_CHW = _C * _HW
_SUB = _CHW // 128
_K2 = 2 * _K
_OFFSET_W_RATIO = 1.0
_OFFSET_H_RATIO = 1.0
_HH = 1.0
_EPS = 1e-07


def _argmax_body(hm_ref, out_ref):
    vc = hm_ref[0]
    cm = jnp.max(jnp.max(vc, axis=1), axis=1)
    m = jnp.max(cm)
    c_iota = lax.broadcasted_iota(jnp.int32, (_C,), 0)
    cstar = jnp.min(jnp.where(cm == m, c_iota, jnp.int32(_C)))
    blk = hm_ref[0, cstar]
    row = lax.broadcasted_iota(jnp.int32, (_H, _W), 0)
    col = lax.broadcasted_iota(jnp.int32, (_H, _W), 1)
    flat = row * _W + col
    p = jnp.min(jnp.where(blk == m, flat, jnp.int32(_HW)))
    out_ref[0] = jnp.full((1, 8), p, jnp.int32)


def _rsqrt(x):
    i = lax.bitcast_convert_type(x, jnp.int32)
    i = jnp.int32(0x5F3759DF) - lax.shift_right_arithmetic(i, 1)
    y = lax.bitcast_convert_type(i, jnp.float32)
    for _ in range(4):
        y = y * (1.5 - 0.5 * x * y * y)
    return y


def _take16(v, idx):
    return lax.gather(
        v, idx[:, None],
        dimension_numbers=lax.GatherDimensionNumbers(
            offset_dims=(), collapsed_slice_dims=(0,), start_index_map=(0,)),
        slice_sizes=(1,),
        mode=lax.GatherScatterMode.PROMISE_IN_BOUNDS)


def _sc_loss_body(wh_hbm, off_hbm, ckp_hbm, tgt_hbm, msk_hbm, p_hbm, i0_hbm,
                  out_hbm,
                  wh_tile, off_tile, ckp_v, tgt_v, msk_v, p_v, i0_v, res_v,
                  sem_p, sem_i, sem_c, sem_t, sem_m, sem_w, sem_o):
    b = lax.axis_index("s") * 2 + lax.axis_index("c")
    lane = lax.broadcasted_iota(jnp.int32, (16,), 0)
    bm = lane == (b % 16)

    cp_p = pltpu.async_copy(p_hbm, p_v, sem_p)
    cp_i = pltpu.async_copy(i0_hbm, i0_v, sem_i)
    cp_c = pltpu.async_copy(ckp_hbm, ckp_v, sem_c)
    cp_t = pltpu.async_copy(tgt_hbm, tgt_v, sem_t)
    cp_m = pltpu.async_copy(msk_hbm, msk_v, sem_m)
    cp_p.wait()
    cp_i.wait()

    def _scalar_i32(ref):
        e0 = jnp.max(jnp.where(bm, ref[pl.ds(0, 16)], jnp.int32(-1)))
        e1 = jnp.max(jnp.where(bm, ref[pl.ds(16, 16)], jnp.int32(-1)))
        return jnp.where(b < 16, e0, e1)

    pp = _scalar_i32(p_v)
    i0 = _scalar_i32(i0_v)
    y1, x1 = i0 // _W, i0 % _W
    y2, x2 = pp // _W, pp % _W
    cp_w = pltpu.async_copy(
        wh_hbm.at[b, :, pl.ds(8 * (y1 // 8), 8), pl.ds(128 * (x1 // 128), 128)],
        wh_tile, sem_w)
    cp_o = pltpu.async_copy(
        off_hbm.at[b, :, pl.ds(8 * (y2 // 8), 8), pl.ds(128 * (x2 // 128), 128)],
        off_tile, sem_o)

    c14 = jnp.minimum(lane, 13)
    bb = jnp.full((16,), b, jnp.int32)
    cp_c.wait()
    cp_t.wait()
    cp_m.wait()
    mskv = plsc.load_gather(msk_v, [bb])
    ckp = plsc.load_gather(ckp_v, [bb, c14]) * mskv
    tgt = plsc.load_gather(tgt_v, [bb, c14]) * mskv
    xs = x2.astype(jnp.float32)
    ys = y2.astype(jnp.float32)
    pos = jnp.where(lane % 2 == 0, jnp.full((16,), xs), jnp.full((16,), ys))
    cp_w.wait()
    cp_o.wait()
    pred = plsc.load_gather(wh_tile, [c14, jnp.full((16,), y1 % 8, jnp.int32),
                                      jnp.full((16,), x1 % 128, jnp.int32)])
    off = plsc.load_gather(off_tile, [c14, jnp.full((16,), y2 % 8, jnp.int32),
                                      jnp.full((16,), x2 % 128, jnp.int32)])
    p1 = pos + off - pred * 0.5
    p2 = pos + off + pred * 0.5
    g1 = ckp - tgt * 0.5
    g2 = ckp + tgt * 0.5
    pc = (p1 + p2) / 2
    gc = (g1 + g2) / 2

    idx2 = jnp.minimum(lane + 2, 15)
    idx1 = jnp.minimum(lane + 1, 15)

    def sh2(v):
        return _take16(v, idx2)

    def pair(v):
        return v + _take16(v, idx1)

    vgg = sh2(gc) - gc
    vpp = sh2(pc) - pc
    vgp = sh2(gc) - pc
    vpg = sh2(pc) - gc
    dp = pc - gc
    dn = sh2(pc) - sh2(gc)

    d2p = pair(dp * dp) + _EPS
    d2n = pair(dn * dn) + _EPS
    n2p = pair(vgp * vgp) + _HH
    n2n = pair(vpg * vpg) + _HH
    rp = _rsqrt(n2p)
    rn = _rsqrt(n2n)
    sin = (d2p * _rsqrt(d2p) * rp + d2n * _rsqrt(d2n) * rn) / 2
    cross = (pair(vgp * vpg) + _HH) * (rp * rn)
    own = (pair(vgg * vpp) + _HH) * (
        _rsqrt(pair(vgg * vgg) + _HH) * _rsqrt(pair(vpp * vpp) + _HH))
    cos = 1.0 - (cross + own) / 2
    gsin = _take16(0.5 * sin, jnp.minimum(lane * 2, 15))
    gcos = _take16(0.5 * cos, jnp.clip((lane - 8) * 2, 0, 15))
    res_v[...] = jnp.where(lane < 8, gsin, gcos)
    pltpu.sync_copy(res_v, out_hbm.at[b])


def kernel(centerKpoints, target_wh, output_hm, output_wh, output_STA_offset,
           mask, index):
    p3 = pl.pallas_call(
        _argmax_body,
        grid=(_B,),
        in_specs=[pl.BlockSpec((1, _C, _H, _W), lambda b: (b, 0, 0, 0))],
        out_specs=pl.BlockSpec((1, 1, 8), lambda b: (b, 0, 0)),
        out_shape=jax.ShapeDtypeStruct((_B, 1, 8), jnp.int32),
    )(output_hm)
    p = p3[:, 0, 0]
    idx0 = index[:, 0].astype(jnp.int32)
    ckp0 = centerKpoints[:, 0, :]
    tgt0 = target_wh[:, 0, :]
    msk0 = mask[:, 0]

    sc_loss = pl.kernel(
        _sc_loss_body,
        out_type=jax.ShapeDtypeStruct((_B, 16), jnp.float32),
        mesh=plsc.VectorSubcoreMesh(core_axis_name="c", subcore_axis_name="s",
                                    num_cores=2, num_subcores=16),
        scratch_types=[
            pltpu.VMEM((_K2, 8, 128), jnp.float32),
            pltpu.VMEM((_K2, 8, 128), jnp.float32),
            pltpu.VMEM((_B, _K2), jnp.float32),
            pltpu.VMEM((_B, _K2), jnp.float32),
            pltpu.VMEM((_B,), jnp.float32),
            pltpu.VMEM((_B,), jnp.int32),
            pltpu.VMEM((_B,), jnp.int32),
            pltpu.VMEM((16,), jnp.float32),
        ] + [pltpu.SemaphoreType.DMA] * 7,
        compiler_params=pltpu.CompilerParams(use_tc_tiling_on_sc=True,
                                             needs_layout_passes=False),
    )
    out = sc_loss(output_wh, output_STA_offset, ckp0, tgt0, msk0, p, idx0)
    sin = out[:, 0:6].reshape(-1)
    cos = out[:, 8:14].reshape(-1)
    return sin, cos
